# baseline (speedup 1.0000x reference)
import functools

import jax
import jax.numpy as jnp
from jax import lax
from jax.experimental import pallas as pl
from jax.experimental.pallas import tpu as pltpu
from jax.experimental.pallas import tpu_sc as plsc

N_NODES = 10000
NP = 10240
D = 128
R = 8
E = 320000
EPAD = 327680
NC = 2
NS = 16
NW = NC * NS
EPW = EPAD // NW
CB = 128
NCHUNK = EPW // CB
G = R * NP
GR = G // CB
RPS = GR // NS
RPB = NP // NS

f32 = jnp.float32
i32 = jnp.int32

_mesh = functools.partial(
    plsc.VectorSubcoreMesh, core_axis_name="c", subcore_axis_name="s",
    num_cores=NC, num_subcores=NS)

_sc_params = pltpu.CompilerParams(needs_layout_passes=False)
_sc_params_nt = pltpu.CompilerParams(needs_layout_passes=False,
                                     use_tc_tiling_on_sc=False)


def _wid():
    return lax.axis_index("s") * NC + lax.axis_index("c")


@functools.partial(
    pl.kernel,
    out_type=jax.ShapeDtypeStruct((NP, D), f32),
    mesh=_mesh(),
    compiler_params=_sc_params,
    scratch_types=[
        pltpu.VMEM((80,), i32),
        pltpu.VMEM((80, D), f32),
        pltpu.SemaphoreType.DMA,
    ],
)
def _emb_gather(emb_hbm, x_hbm, out_hbm, idx_v, rows_v, sem):
    base = _wid() * (NP // NW)
    for j in range(NP // NW // 80):
        off = base + 80 * j
        pltpu.sync_copy(x_hbm.at[pl.ds(off, 80)], idx_v)
        pltpu.async_copy(emb_hbm.at[idx_v], rows_v, sem).wait()
        pltpu.sync_copy(rows_v, out_hbm.at[pl.ds(off, 80)])


CB0 = 2048


@functools.partial(
    pl.kernel,
    out_type=jax.ShapeDtypeStruct((NW, G), f32),
    mesh=_mesh(),
    compiler_params=_sc_params,
    scratch_types=[
        pltpu.VMEM((G,), f32),
        pltpu.VMEM((CB0,), i32),
        pltpu.VMEM((CB0,), i32),
    ],
)
def _edge_hist(dst_hbm, rel_hbm, hist_hbm, cnt_v, db, rb):
    wid = _wid()
    ones16 = jnp.ones((16,), f32)
    zeros16 = jnp.zeros((16,), f32)

    def zloc(t, _):
        cnt_v[pl.ds(t * 16, 16)] = zeros16
        return 0
    lax.fori_loop(0, G // 16, zloc, 0)

    base_w = wid * EPW

    def cbody(ci, _):
        b = base_w + ci * CB0
        pltpu.sync_copy(dst_hbm.at[pl.ds(b, CB0)], db)
        pltpu.sync_copy(rel_hbm.at[pl.ds(b, CB0)], rb)

        def ib(k, _):
            off = k * 16
            g16 = rb[pl.ds(off, 16)] * NP + db[pl.ds(off, 16)]
            plsc.addupdate_scatter(cnt_v, [g16], ones16)
            return 0
        lax.fori_loop(0, CB0 // 16, ib, 0)
        return 0
    lax.fori_loop(0, EPW // CB0, cbody, 0)
    pltpu.sync_copy(cnt_v, hist_hbm.at[wid])


GB = 8192


def _inv_body(hist_ref, inv_ref):
    s = jnp.sum(hist_ref[...], axis=0, keepdims=True)
    inv_ref[...] = 1.0 / jnp.maximum(s, 1.0)


_inv_cnt = pl.pallas_call(
    _inv_body,
    grid=(G // GB,),
    in_specs=[pl.BlockSpec((NW, GB), lambda i: (0, i))],
    out_specs=pl.BlockSpec((1, GB), lambda i: (0, i)),
    out_shape=jax.ShapeDtypeStruct((1, G), f32),
)


@functools.partial(
    pl.kernel,
    out_type=(jax.ShapeDtypeStruct((EPAD,), f32),
              jax.ShapeDtypeStruct((EPAD,), i32)),
    mesh=_mesh(),
    compiler_params=_sc_params,
    scratch_types=[
        pltpu.VMEM((G,), f32),
        pltpu.VMEM((CB0,), i32),
        pltpu.VMEM((CB0,), i32),
        pltpu.VMEM((CB0,), i32),
        pltpu.VMEM((CB0,), f32),
        pltpu.VMEM((CB0,), i32),
    ],
)
def _edge_scale(src_hbm, dst_hbm, rel_hbm, inv_hbm, scale_hbm, gidx_hbm,
                inv_v, sb, db, rb, scv, gsv):
    wid = _wid()
    iota16 = lax.iota(i32, 16)
    pltpu.sync_copy(inv_hbm, inv_v)
    base_w = wid * EPW

    def cbody(ci, _):
        b = base_w + ci * CB0
        pltpu.sync_copy(src_hbm.at[pl.ds(b, CB0)], sb)
        pltpu.sync_copy(dst_hbm.at[pl.ds(b, CB0)], db)
        pltpu.sync_copy(rel_hbm.at[pl.ds(b, CB0)], rb)

        def ib(k, _):
            off = k * 16
            r16 = rb[pl.ds(off, 16)]
            g16 = r16 * NP + db[pl.ds(off, 16)]
            sc16 = plsc.load_gather(inv_v, [g16])
            eid = b + off + iota16
            scv[pl.ds(off, 16)] = jnp.where(eid < E, sc16, 0.0)
            gsv[pl.ds(off, 16)] = r16 * NP + sb[pl.ds(off, 16)]
            return 0
        lax.fori_loop(0, CB0 // 16, ib, 0)
        pltpu.sync_copy(scv, scale_hbm.at[pl.ds(b, CB0)])
        pltpu.sync_copy(gsv, gidx_hbm.at[pl.ds(b, CB0)])
        return 0
    lax.fori_loop(0, EPW // CB0, cbody, 0)


WCH = 16
C0 = 48
C1 = 160 - C0


@functools.partial(
    pl.kernel,
    out_type=jax.ShapeDtypeStruct((NC, NP, D), f32),
    mesh=_mesh(),
    compiler_params=_sc_params,
    scratch_types=[
        pltpu.VMEM((WCH * CB,), i32),
        pltpu.VMEM((WCH * CB,), i32),
        pltpu.VMEM((WCH * CB,), f32),
        pltpu.VMEM((CB,), i32),
        pltpu.VMEM((CB,), i32),
        pltpu.VMEM((CB,), i32),
        pltpu.VMEM((CB, D), f32),
        pltpu.VMEM((CB, D), f32),
        pltpu.VMEM_SHARED((NP, D), f32),
        pltpu.SemaphoreType.DMA,
        pltpu.SemaphoreType.DMA,
    ],
)
def _msg_pass(hw_hbm, gidx_hbm, dst_hbm, scale_hbm, out_hbm,
              ga, da, scv, gc0, gc1, dc, rows0, rows1, acc, sem0, sem1):
    cid = lax.axis_index("c")
    sid = lax.axis_index("s")
    zeros16 = jnp.zeros((16,), f32)
    rows = (rows0, rows1)
    gcs = (gc0, gc1)
    sems = (sem0, sem1)

    def zrow(t, _):
        for k in range(D // 16):
            rows0[t, pl.ds(k * 16, 16)] = zeros16
        return 0
    lax.fori_loop(0, CB, zrow, 0)
    for j in range(RPB // CB):
        pltpu.sync_copy(rows0, acc.at[pl.ds(sid * RPB + j * CB, CB)])
    plsc.subcore_barrier()

    def fill(dst_ref, src_ref, lc):
        for k in range(CB // 16):
            dst_ref[pl.ds(k * 16, 16)] = src_ref[pl.ds(lc * CB + k * 16, 16)]

    def do_chunk(lc, b):
        nxt = lc + 1

        @pl.when(nxt < WCH)
        def _():
            fill(gcs[1 - b], ga, nxt)
            pltpu.async_copy(hw_hbm.at[gcs[1 - b]], rows[1 - b],
                             sems[1 - b])
        pltpu.make_async_copy(hw_hbm.at[gcs[b]], rows[b], sems[b]).wait()
        sbase = lc * CB

        def srow(j, _):
            sj = plsc.load_gather(scv, [jnp.full((16,), sbase + j, i32)])
            for k in range(D // 16):
                rows[b][j, pl.ds(16 * k, 16)] = (
                    rows[b][j, pl.ds(16 * k, 16)] * sj)
            return 0
        lax.fori_loop(0, CB, srow, 0)
        fill(dc, da, lc)
        pltpu.sync_copy(rows[b], acc.at[dc], add=True)

    base_e = (sid * (C0 + C1) + cid * C0) * CB
    nwin = (C0 - (C0 - C1) * cid) // WCH

    def window(w, _):
        eb = base_e + w * (WCH * CB)
        pltpu.sync_copy(gidx_hbm.at[pl.ds(eb, WCH * CB)], ga)
        pltpu.sync_copy(dst_hbm.at[pl.ds(eb, WCH * CB)], da)
        pltpu.sync_copy(scale_hbm.at[pl.ds(eb, WCH * CB)], scv)
        fill(gc0, ga, 0)
        pltpu.async_copy(hw_hbm.at[gc0], rows0, sem0)

        def pair(i, _):
            do_chunk(2 * i, 0)
            do_chunk(2 * i + 1, 1)
            return 0
        lax.fori_loop(0, WCH // 2, pair, 0)
        return 0
    lax.fori_loop(0, nwin, window, 0)
    plsc.subcore_barrier()

    for j in range(RPB // CB):
        rs = sid * RPB + j * CB
        pltpu.sync_copy(acc.at[pl.ds(rs, CB)], out_hbm.at[cid, pl.ds(rs, CB)])

BD = 1024


def _dense_body(h_ref, root_ref, b_ref, w_ref, out0_ref, hw_ref):
    x = h_ref[...]
    out0_ref[...] = (
        jnp.dot(x, root_ref[...], preferred_element_type=f32) + b_ref[...])
    for r in range(R):
        hw_ref[r] = jnp.dot(x, w_ref[r], preferred_element_type=f32)


_dense = pl.pallas_call(
    _dense_body,
    grid=(NP // BD,),
    in_specs=[
        pl.BlockSpec((BD, D), lambda i: (i, 0)),
        pl.BlockSpec((D, D), lambda i: (0, 0)),
        pl.BlockSpec((1, D), lambda i: (0, 0)),
        pl.BlockSpec((R, D, D), lambda i: (0, 0, 0)),
    ],
    out_specs=[
        pl.BlockSpec((BD, D), lambda i: (i, 0)),
        pl.BlockSpec((R, BD, D), lambda i: (0, i, 0)),
    ],
    out_shape=[
        jax.ShapeDtypeStruct((NP, D), f32),
        jax.ShapeDtypeStruct((R, NP, D), f32),
    ],
)


def _combine_body(out0_ref, msg_ref, g_ref, be_ref, o_ref, *, ln):
    h = out0_ref[...] + msg_ref[0] + msg_ref[1]
    if ln:
        mu = jnp.mean(h, axis=-1, keepdims=True)
        var = jnp.mean((h - mu) ** 2, axis=-1, keepdims=True)
        h = (h - mu) / jnp.sqrt(var + 1e-5) * g_ref[...] + be_ref[...]
        h = jnp.maximum(h, 0.0)
    o_ref[...] = h


def _make_combine(ln):
    return pl.pallas_call(
        functools.partial(_combine_body, ln=ln),
        grid=(NP // BD,),
        in_specs=[
            pl.BlockSpec((BD, D), lambda i: (i, 0)),
            pl.BlockSpec((NC, BD, D), lambda i: (0, i, 0)),
            pl.BlockSpec((1, D), lambda i: (0, 0)),
            pl.BlockSpec((1, D), lambda i: (0, 0)),
        ],
        out_specs=pl.BlockSpec((BD, D), lambda i: (i, 0)),
        out_shape=jax.ShapeDtypeStruct((NP, D), f32),
    )


_combine_ln = _make_combine(True)
_combine_plain = _make_combine(False)


def kernel(x, edge_index, edge_type, emb_table, W1, root1, b1, g1, be1,
           W2, root2, b2, g2, be2, W3, root3, b3):
    xi = x.astype(i32)
    src = edge_index[0].astype(i32)
    dst = edge_index[1].astype(i32)
    rel = edge_type.astype(i32)

    xp = jnp.concatenate([xi, jnp.zeros((NP - N_NODES,), i32)])
    padE = EPAD - E
    zpad = jnp.zeros((padE,), i32)
    srcp = jnp.concatenate([src, zpad])
    dstp = jnp.concatenate([dst, jnp.full((padE,), N_NODES, i32)])
    relp = jnp.concatenate([rel, zpad])

    h = _emb_gather(emb_table, xp)
    hist = _edge_hist(dstp, relp)
    inv = _inv_cnt(hist)
    scale, gidx = _edge_scale(srcp, dstp, relp, inv.reshape(G))

    layers = [
        (W1, root1, b1, g1, be1, True),
        (W2, root2, b2, g2, be2, True),
        (W3, root3, b3, None, None, False),
    ]
    for (Wl, rootl, bl, gl, bel, ln) in layers:
        out0, hw = _dense(h, rootl, bl.reshape(1, D), Wl)
        msgs = _msg_pass(hw.reshape(R * NP, D), gidx, dstp, scale)
        if ln:
            h = _combine_ln(out0, msgs, gl.reshape(1, D), bel.reshape(1, D))
        else:
            h = _combine_plain(out0, msgs, bl.reshape(1, D),
                               bl.reshape(1, D))
    return h[:N_NODES]


# speedup vs baseline: 1.0948x; 1.0948x over previous
import functools

import jax
import jax.numpy as jnp
from jax import lax
from jax.experimental import pallas as pl
from jax.experimental.pallas import tpu as pltpu
from jax.experimental.pallas import tpu_sc as plsc

N_NODES = 10000
NP = 10240
D = 128
R = 8
E = 320000
EPAD = 327680
NC = 2
NS = 16
NW = NC * NS
EPW = EPAD // NW
CB = 128
NCHUNK = EPW // CB
G = R * NP
GR = G // CB
RPS = GR // NS
RPB = NP // NS

f32 = jnp.float32
i32 = jnp.int32

_mesh = functools.partial(
    plsc.VectorSubcoreMesh, core_axis_name="c", subcore_axis_name="s",
    num_cores=NC, num_subcores=NS)

_sc_params = pltpu.CompilerParams(needs_layout_passes=False)
_sc_params_nt = pltpu.CompilerParams(needs_layout_passes=False,
                                     use_tc_tiling_on_sc=False)


def _wid():
    return lax.axis_index("s") * NC + lax.axis_index("c")


@functools.partial(
    pl.kernel,
    out_type=jax.ShapeDtypeStruct((NP, D), f32),
    mesh=_mesh(),
    compiler_params=_sc_params,
    scratch_types=[
        pltpu.VMEM((80,), i32),
        pltpu.VMEM((80, D), f32),
        pltpu.SemaphoreType.DMA,
    ],
)
def _emb_gather(emb_hbm, x_hbm, out_hbm, idx_v, rows_v, sem):
    base = _wid() * (NP // NW)
    for j in range(NP // NW // 80):
        off = base + 80 * j
        pltpu.sync_copy(x_hbm.at[pl.ds(off, 80)], idx_v)
        pltpu.async_copy(emb_hbm.at[idx_v], rows_v, sem).wait()
        pltpu.sync_copy(rows_v, out_hbm.at[pl.ds(off, 80)])


CB0 = 2048


@functools.partial(
    pl.kernel,
    out_type=jax.ShapeDtypeStruct((NW, G), f32),
    mesh=_mesh(),
    compiler_params=_sc_params,
    scratch_types=[
        pltpu.VMEM((G,), f32),
        pltpu.VMEM((CB0,), i32),
        pltpu.VMEM((CB0,), i32),
    ],
)
def _edge_hist(dst_hbm, rel_hbm, hist_hbm, cnt_v, db, rb):
    wid = _wid()
    ones16 = jnp.ones((16,), f32)
    zeros16 = jnp.zeros((16,), f32)

    def zloc(t, _):
        cnt_v[pl.ds(t * 16, 16)] = zeros16
        return 0
    lax.fori_loop(0, G // 16, zloc, 0)

    base_w = wid * EPW

    def cbody(ci, _):
        b = base_w + ci * CB0
        pltpu.sync_copy(dst_hbm.at[pl.ds(b, CB0)], db)
        pltpu.sync_copy(rel_hbm.at[pl.ds(b, CB0)], rb)

        def ib(k, _):
            off = k * 16
            g16 = rb[pl.ds(off, 16)] * NP + db[pl.ds(off, 16)]
            plsc.addupdate_scatter(cnt_v, [g16], ones16)
            return 0
        lax.fori_loop(0, CB0 // 16, ib, 0)
        return 0
    lax.fori_loop(0, EPW // CB0, cbody, 0)
    pltpu.sync_copy(cnt_v, hist_hbm.at[wid])


GB = 8192


def _inv_body(hist_ref, inv_ref):
    s = jnp.sum(hist_ref[...], axis=0, keepdims=True)
    inv_ref[...] = 1.0 / jnp.maximum(s, 1.0)


_inv_cnt = pl.pallas_call(
    _inv_body,
    grid=(G // GB,),
    in_specs=[pl.BlockSpec((NW, GB), lambda i: (0, i))],
    out_specs=pl.BlockSpec((1, GB), lambda i: (0, i)),
    out_shape=jax.ShapeDtypeStruct((1, G), f32),
)


@functools.partial(
    pl.kernel,
    out_type=(jax.ShapeDtypeStruct((EPAD,), f32),
              jax.ShapeDtypeStruct((EPAD,), i32)),
    mesh=_mesh(),
    compiler_params=_sc_params,
    scratch_types=[
        pltpu.VMEM((G,), f32),
        pltpu.VMEM((CB0,), i32),
        pltpu.VMEM((CB0,), i32),
        pltpu.VMEM((CB0,), i32),
        pltpu.VMEM((CB0,), f32),
        pltpu.VMEM((CB0,), i32),
    ],
)
def _edge_scale(src_hbm, dst_hbm, rel_hbm, inv_hbm, scale_hbm, gidx_hbm,
                inv_v, sb, db, rb, scv, gsv):
    wid = _wid()
    iota16 = lax.iota(i32, 16)
    pltpu.sync_copy(inv_hbm, inv_v)
    base_w = wid * EPW

    def cbody(ci, _):
        b = base_w + ci * CB0
        pltpu.sync_copy(src_hbm.at[pl.ds(b, CB0)], sb)
        pltpu.sync_copy(dst_hbm.at[pl.ds(b, CB0)], db)
        pltpu.sync_copy(rel_hbm.at[pl.ds(b, CB0)], rb)

        def ib(k, _):
            off = k * 16
            r16 = rb[pl.ds(off, 16)]
            g16 = r16 * NP + db[pl.ds(off, 16)]
            sc16 = plsc.load_gather(inv_v, [g16])
            eid = b + off + iota16
            scv[pl.ds(off, 16)] = jnp.where(eid < E, sc16, 0.0)
            gsv[pl.ds(off, 16)] = r16 * NP + sb[pl.ds(off, 16)]
            return 0
        lax.fori_loop(0, CB0 // 16, ib, 0)
        pltpu.sync_copy(scv, scale_hbm.at[pl.ds(b, CB0)])
        pltpu.sync_copy(gsv, gidx_hbm.at[pl.ds(b, CB0)])
        return 0
    lax.fori_loop(0, EPW // CB0, cbody, 0)


WCH = 16
C0 = 80
C1 = 160 - C0


@functools.partial(
    pl.kernel,
    out_type=jax.ShapeDtypeStruct((NC, NP, D), f32),
    mesh=_mesh(),
    compiler_params=_sc_params,
    scratch_types=[
        pltpu.VMEM((WCH * CB,), i32),
        pltpu.VMEM((WCH * CB,), i32),
        pltpu.VMEM((WCH * CB,), f32),
        pltpu.VMEM((CB,), i32),
        pltpu.VMEM((CB,), i32),
        pltpu.VMEM((CB,), i32),
        pltpu.VMEM((CB,), i32),
        pltpu.VMEM((CB, D), f32),
        pltpu.VMEM((CB, D), f32),
        pltpu.VMEM_SHARED((NP, D), f32),
        pltpu.SemaphoreType.DMA,
        pltpu.SemaphoreType.DMA,
        pltpu.SemaphoreType.DMA,
        pltpu.SemaphoreType.DMA,
    ],
)
def _msg_pass(hw_hbm, gidx_hbm, dst_hbm, scale_hbm, out_hbm,
              ga, da, scv, gc0, gc1, dc0, dc1, rows0, rows1, acc,
              sem0, sem1, ssem0, ssem1):
    cid = lax.axis_index("c")
    sid = lax.axis_index("s")
    zeros16 = jnp.zeros((16,), f32)
    rows = (rows0, rows1)
    gcs = (gc0, gc1)
    dcs = (dc0, dc1)
    sems = (sem0, sem1)
    ssems = (ssem0, ssem1)

    def zrow(t, _):
        for k in range(D // 16):
            rows0[t, pl.ds(k * 16, 16)] = zeros16
        return 0
    lax.fori_loop(0, CB, zrow, 0)
    for j in range(RPB // CB):
        pltpu.sync_copy(rows0, acc.at[pl.ds(sid * RPB + j * CB, CB)])
    plsc.subcore_barrier()

    def fill(dst_ref, src_ref, lc):
        for k in range(CB // 16):
            dst_ref[pl.ds(k * 16, 16)] = src_ref[pl.ds(lc * CB + k * 16, 16)]

    def do_chunk(lc, b):
        nxt = lc + 1

        @pl.when(nxt < WCH)
        def _():
            fill(gcs[1 - b], ga, nxt)

            @pl.when(lc >= 1)
            def _():
                pltpu.make_async_copy(rows[1 - b], acc.at[dcs[1 - b]],
                                      ssems[1 - b]).wait()
            pltpu.async_copy(hw_hbm.at[gcs[1 - b]], rows[1 - b],
                             sems[1 - b])
        pltpu.make_async_copy(hw_hbm.at[gcs[b]], rows[b], sems[b]).wait()
        sbase = lc * CB

        def srow(j, _):
            sj = plsc.load_gather(scv, [jnp.full((16,), sbase + j, i32)])
            for k in range(D // 16):
                rows[b][j, pl.ds(16 * k, 16)] = (
                    rows[b][j, pl.ds(16 * k, 16)] * sj)
            return 0
        lax.fori_loop(0, CB, srow, 0)
        fill(dcs[b], da, lc)

        @pl.when(lc < WCH - 2)
        def _():
            pltpu.async_copy(rows[b], acc.at[dcs[b]], ssems[b], add=True)

        @pl.when(lc >= WCH - 2)
        def _():
            pltpu.sync_copy(rows[b], acc.at[dcs[b]], add=True)

    base_e = (sid * (C0 + C1) + cid * C0) * CB
    nwin = (C0 - (C0 - C1) * cid) // WCH

    def window(w, _):
        eb = base_e + w * (WCH * CB)
        pltpu.sync_copy(gidx_hbm.at[pl.ds(eb, WCH * CB)], ga)
        pltpu.sync_copy(dst_hbm.at[pl.ds(eb, WCH * CB)], da)
        pltpu.sync_copy(scale_hbm.at[pl.ds(eb, WCH * CB)], scv)
        fill(gc0, ga, 0)
        pltpu.async_copy(hw_hbm.at[gc0], rows0, sem0)

        def pair(i, _):
            do_chunk(2 * i, 0)
            do_chunk(2 * i + 1, 1)
            return 0
        lax.fori_loop(0, WCH // 2, pair, 0)
        return 0
    lax.fori_loop(0, nwin, window, 0)
    plsc.subcore_barrier()

    for j in range(RPB // CB):
        rs = sid * RPB + j * CB
        pltpu.sync_copy(acc.at[pl.ds(rs, CB)], out_hbm.at[cid, pl.ds(rs, CB)])

BD = 1024


def _dense_body(h_ref, root_ref, b_ref, w_ref, out0_ref, hw_ref):
    x = h_ref[...]
    out0_ref[...] = (
        jnp.dot(x, root_ref[...], preferred_element_type=f32) + b_ref[...])
    for r in range(R):
        hw_ref[r] = jnp.dot(x, w_ref[r], preferred_element_type=f32)


_dense = pl.pallas_call(
    _dense_body,
    grid=(NP // BD,),
    in_specs=[
        pl.BlockSpec((BD, D), lambda i: (i, 0)),
        pl.BlockSpec((D, D), lambda i: (0, 0)),
        pl.BlockSpec((1, D), lambda i: (0, 0)),
        pl.BlockSpec((R, D, D), lambda i: (0, 0, 0)),
    ],
    out_specs=[
        pl.BlockSpec((BD, D), lambda i: (i, 0)),
        pl.BlockSpec((R, BD, D), lambda i: (0, i, 0)),
    ],
    out_shape=[
        jax.ShapeDtypeStruct((NP, D), f32),
        jax.ShapeDtypeStruct((R, NP, D), f32),
    ],
)


def _combine_body(out0_ref, msg_ref, g_ref, be_ref, o_ref, *, ln):
    h = out0_ref[...] + msg_ref[0] + msg_ref[1]
    if ln:
        mu = jnp.mean(h, axis=-1, keepdims=True)
        var = jnp.mean((h - mu) ** 2, axis=-1, keepdims=True)
        h = (h - mu) / jnp.sqrt(var + 1e-5) * g_ref[...] + be_ref[...]
        h = jnp.maximum(h, 0.0)
    o_ref[...] = h


def _make_combine(ln):
    return pl.pallas_call(
        functools.partial(_combine_body, ln=ln),
        grid=(NP // BD,),
        in_specs=[
            pl.BlockSpec((BD, D), lambda i: (i, 0)),
            pl.BlockSpec((NC, BD, D), lambda i: (0, i, 0)),
            pl.BlockSpec((1, D), lambda i: (0, 0)),
            pl.BlockSpec((1, D), lambda i: (0, 0)),
        ],
        out_specs=pl.BlockSpec((BD, D), lambda i: (i, 0)),
        out_shape=jax.ShapeDtypeStruct((NP, D), f32),
    )


_combine_ln = _make_combine(True)
_combine_plain = _make_combine(False)


def kernel(x, edge_index, edge_type, emb_table, W1, root1, b1, g1, be1,
           W2, root2, b2, g2, be2, W3, root3, b3):
    xi = x.astype(i32)
    src = edge_index[0].astype(i32)
    dst = edge_index[1].astype(i32)
    rel = edge_type.astype(i32)

    xp = jnp.concatenate([xi, jnp.zeros((NP - N_NODES,), i32)])
    padE = EPAD - E
    zpad = jnp.zeros((padE,), i32)
    srcp = jnp.concatenate([src, zpad])
    dstp = jnp.concatenate([dst, jnp.full((padE,), N_NODES, i32)])
    relp = jnp.concatenate([rel, zpad])

    h = _emb_gather(emb_table, xp)
    hist = _edge_hist(dstp, relp)
    inv = _inv_cnt(hist)
    scale, gidx = _edge_scale(srcp, dstp, relp, inv.reshape(G))

    layers = [
        (W1, root1, b1, g1, be1, True),
        (W2, root2, b2, g2, be2, True),
        (W3, root3, b3, None, None, False),
    ]
    for (Wl, rootl, bl, gl, bel, ln) in layers:
        out0, hw = _dense(h, rootl, bl.reshape(1, D), Wl)
        msgs = _msg_pass(hw.reshape(R * NP, D), gidx, dstp, scale)
        if ln:
            h = _combine_ln(out0, msgs, gl.reshape(1, D), bel.reshape(1, D))
        else:
            h = _combine_plain(out0, msgs, bl.reshape(1, D),
                               bl.reshape(1, D))
    return h[:N_NODES]


# speedup vs baseline: 1.2308x; 1.1242x over previous
import functools

import jax
import jax.numpy as jnp
from jax import lax
from jax.experimental import pallas as pl
from jax.experimental.pallas import tpu as pltpu
from jax.experimental.pallas import tpu_sc as plsc

N_NODES = 10000
NP = 10240
D = 128
R = 8
E = 320000
EPAD = 327680
NC = 2
NS = 16
NW = NC * NS
EPW = EPAD // NW
CB = 128
NCHUNK = EPW // CB
G = R * NP
GR = G // CB
RPS = GR // NS
RPB = NP // NS

f32 = jnp.float32
i32 = jnp.int32

_mesh = functools.partial(
    plsc.VectorSubcoreMesh, core_axis_name="c", subcore_axis_name="s",
    num_cores=NC, num_subcores=NS)

_sc_params = pltpu.CompilerParams(needs_layout_passes=False)
_sc_params_nt = pltpu.CompilerParams(needs_layout_passes=False,
                                     use_tc_tiling_on_sc=False)


def _wid():
    return lax.axis_index("s") * NC + lax.axis_index("c")


@functools.partial(
    pl.kernel,
    out_type=jax.ShapeDtypeStruct((NP, D), f32),
    mesh=_mesh(),
    compiler_params=_sc_params,
    scratch_types=[
        pltpu.VMEM((80,), i32),
        pltpu.VMEM((80, D), f32),
        pltpu.SemaphoreType.DMA,
    ],
)
def _emb_gather(emb_hbm, x_hbm, out_hbm, idx_v, rows_v, sem):
    base = _wid() * (NP // NW)
    for j in range(NP // NW // 80):
        off = base + 80 * j
        pltpu.sync_copy(x_hbm.at[pl.ds(off, 80)], idx_v)
        pltpu.async_copy(emb_hbm.at[idx_v], rows_v, sem).wait()
        pltpu.sync_copy(rows_v, out_hbm.at[pl.ds(off, 80)])


CB0 = 2048


@functools.partial(
    pl.kernel,
    out_type=jax.ShapeDtypeStruct((NW, G), f32),
    mesh=_mesh(),
    compiler_params=_sc_params,
    scratch_types=[
        pltpu.VMEM((G,), f32),
        pltpu.VMEM((CB0,), i32),
        pltpu.VMEM((CB0,), i32),
    ],
)
def _edge_hist(dst_hbm, rel_hbm, hist_hbm, cnt_v, db, rb):
    wid = _wid()
    ones16 = jnp.ones((16,), f32)
    zeros16 = jnp.zeros((16,), f32)

    def zloc(t, _):
        cnt_v[pl.ds(t * 16, 16)] = zeros16
        return 0
    lax.fori_loop(0, G // 16, zloc, 0)

    base_w = wid * EPW

    def cbody(ci, _):
        b = base_w + ci * CB0
        pltpu.sync_copy(dst_hbm.at[pl.ds(b, CB0)], db)
        pltpu.sync_copy(rel_hbm.at[pl.ds(b, CB0)], rb)

        def ib(k, _):
            off = k * 16
            g16 = rb[pl.ds(off, 16)] * NP + db[pl.ds(off, 16)]
            plsc.addupdate_scatter(cnt_v, [g16], ones16)
            return 0
        lax.fori_loop(0, CB0 // 16, ib, 0)
        return 0
    lax.fori_loop(0, EPW // CB0, cbody, 0)
    pltpu.sync_copy(cnt_v, hist_hbm.at[wid])


GB = 8192


def _inv_body(hist_ref, inv_ref):
    s = jnp.sum(hist_ref[...], axis=0, keepdims=True)
    inv_ref[...] = 1.0 / jnp.maximum(s, 1.0)


_inv_cnt = pl.pallas_call(
    _inv_body,
    grid=(G // GB,),
    in_specs=[pl.BlockSpec((NW, GB), lambda i: (0, i))],
    out_specs=pl.BlockSpec((1, GB), lambda i: (0, i)),
    out_shape=jax.ShapeDtypeStruct((1, G), f32),
)


@functools.partial(
    pl.kernel,
    out_type=(jax.ShapeDtypeStruct((EPAD,), f32),
              jax.ShapeDtypeStruct((EPAD,), i32)),
    mesh=_mesh(),
    compiler_params=_sc_params,
    scratch_types=[
        pltpu.VMEM((G,), f32),
        pltpu.VMEM((CB0,), i32),
        pltpu.VMEM((CB0,), i32),
        pltpu.VMEM((CB0,), i32),
        pltpu.VMEM((CB0,), f32),
        pltpu.VMEM((CB0,), i32),
    ],
)
def _edge_scale(src_hbm, dst_hbm, rel_hbm, inv_hbm, scale_hbm, gidx_hbm,
                inv_v, sb, db, rb, scv, gsv):
    wid = _wid()
    iota16 = lax.iota(i32, 16)
    pltpu.sync_copy(inv_hbm, inv_v)
    base_w = wid * EPW

    def cbody(ci, _):
        b = base_w + ci * CB0
        pltpu.sync_copy(src_hbm.at[pl.ds(b, CB0)], sb)
        pltpu.sync_copy(dst_hbm.at[pl.ds(b, CB0)], db)
        pltpu.sync_copy(rel_hbm.at[pl.ds(b, CB0)], rb)

        def ib(k, _):
            off = k * 16
            r16 = rb[pl.ds(off, 16)]
            g16 = r16 * NP + db[pl.ds(off, 16)]
            sc16 = plsc.load_gather(inv_v, [g16])
            eid = b + off + iota16
            scv[pl.ds(off, 16)] = jnp.where(eid < E, sc16, 0.0)
            gsv[pl.ds(off, 16)] = r16 * NP + sb[pl.ds(off, 16)]
            return 0
        lax.fori_loop(0, CB0 // 16, ib, 0)
        pltpu.sync_copy(scv, scale_hbm.at[pl.ds(b, CB0)])
        pltpu.sync_copy(gsv, gidx_hbm.at[pl.ds(b, CB0)])
        return 0
    lax.fori_loop(0, EPW // CB0, cbody, 0)


DW = D // 2
WCH = 16
C0 = 80
C1 = 160 - C0


@functools.partial(
    pl.kernel,
    out_type=jax.ShapeDtypeStruct((NC, NP, D), f32),
    mesh=_mesh(),
    compiler_params=_sc_params_nt,
    scratch_types=[
        pltpu.VMEM((WCH * CB,), i32),
        pltpu.VMEM((WCH * CB,), i32),
        pltpu.VMEM((WCH * CB,), f32),
        pltpu.VMEM((CB,), i32),
        pltpu.VMEM((CB,), i32),
        pltpu.VMEM((CB,), i32),
        pltpu.VMEM((CB,), i32),
        pltpu.VMEM((CB, DW), i32),
        pltpu.VMEM((CB, DW), i32),
        pltpu.VMEM((CB, D), f32),
        pltpu.VMEM_SHARED((NP, D), f32),
        pltpu.SemaphoreType.DMA,
        pltpu.SemaphoreType.DMA,
    ],
)
def _msg_pass(hw_hbm, gidx_hbm, dst_hbm, scale_hbm, out_hbm,
              ga, da, scv, gc0, gc1, dc0, dc1, brows0, brows1, srows, acc,
              sem0, sem1):
    cid = lax.axis_index("c")
    sid = lax.axis_index("s")
    zeros16 = jnp.zeros((16,), f32)
    himask = jnp.full((16,), -65536, i32)
    brows = (brows0, brows1)
    gcs = (gc0, gc1)
    dcs = (dc0, dc1)
    sems = (sem0, sem1)

    def zrow(t, _):
        for k in range(D // 16):
            srows[t, pl.ds(k * 16, 16)] = zeros16
        return 0
    lax.fori_loop(0, CB, zrow, 0)
    for j in range(RPB // CB):
        pltpu.sync_copy(srows, acc.at[pl.ds(sid * RPB + j * CB, CB)])
    plsc.subcore_barrier()

    def fill(dst_ref, src_ref, lc):
        for k in range(CB // 16):
            dst_ref[pl.ds(k * 16, 16)] = src_ref[pl.ds(lc * CB + k * 16, 16)]

    def do_chunk(lc, b):
        nxt = lc + 1

        @pl.when(nxt < WCH)
        def _():
            fill(gcs[1 - b], ga, nxt)
            pltpu.async_copy(hw_hbm.at[gcs[1 - b]], brows[1 - b],
                             sems[1 - b])
        pltpu.make_async_copy(hw_hbm.at[gcs[b]], brows[b], sems[b]).wait()
        sbase = lc * CB

        def srow(j, _):
            sj = plsc.load_gather(scv, [jnp.full((16,), sbase + j, i32)])
            for k in range(DW // 16):
                v = brows[b][j, pl.ds(16 * k, 16)]
                lo = plsc.bitcast(v << 16, f32)
                hi = plsc.bitcast(v & himask, f32)
                srows[j, pl.ds(16 * k, 16)] = lo * sj
                srows[j, pl.ds(64 + 16 * k, 16)] = hi * sj
            return 0
        lax.fori_loop(0, CB, srow, 0)
        fill(dcs[b], da, lc)
        pltpu.sync_copy(srows, acc.at[dcs[b]], add=True)

    base_e = (sid * (C0 + C1) + cid * C0) * CB
    nwin = (C0 - (C0 - C1) * cid) // WCH

    def window(w, _):
        eb = base_e + w * (WCH * CB)
        pltpu.sync_copy(gidx_hbm.at[pl.ds(eb, WCH * CB)], ga)
        pltpu.sync_copy(dst_hbm.at[pl.ds(eb, WCH * CB)], da)
        pltpu.sync_copy(scale_hbm.at[pl.ds(eb, WCH * CB)], scv)
        fill(gc0, ga, 0)
        pltpu.async_copy(hw_hbm.at[gc0], brows0, sem0)

        def pair(i, _):
            do_chunk(2 * i, 0)
            do_chunk(2 * i + 1, 1)
            return 0
        lax.fori_loop(0, WCH // 2, pair, 0)
        return 0
    lax.fori_loop(0, nwin, window, 0)
    plsc.subcore_barrier()

    for j in range(RPB // CB):
        rs = sid * RPB + j * CB
        pltpu.sync_copy(acc.at[pl.ds(rs, CB)], out_hbm.at[cid, pl.ds(rs, CB)])

BD = 1024


def _dense_body(h_ref, root_ref, b_ref, w_ref, out0_ref, hw_ref):
    x = h_ref[...]
    out0_ref[...] = (
        jnp.dot(x, root_ref[...], preferred_element_type=f32) + b_ref[...])
    for r in range(R):
        y = jnp.dot(x, w_ref[r], preferred_element_type=f32)
        yb = y.astype(jnp.bfloat16).astype(f32)
        bits = lax.bitcast_convert_type(yb, jnp.uint32)
        word = (bits[:, :DW] >> 16) | (bits[:, DW:] & jnp.uint32(0xFFFF0000))
        hw_ref[r] = lax.bitcast_convert_type(word, i32)


_dense = pl.pallas_call(
    _dense_body,
    grid=(NP // BD,),
    in_specs=[
        pl.BlockSpec((BD, D), lambda i: (i, 0)),
        pl.BlockSpec((D, D), lambda i: (0, 0)),
        pl.BlockSpec((1, D), lambda i: (0, 0)),
        pl.BlockSpec((R, D, D), lambda i: (0, 0, 0)),
    ],
    out_specs=[
        pl.BlockSpec((BD, D), lambda i: (i, 0)),
        pl.BlockSpec((R, BD, DW), lambda i: (0, i, 0)),
    ],
    out_shape=[
        jax.ShapeDtypeStruct((NP, D), f32),
        jax.ShapeDtypeStruct((R, NP, DW), i32),
    ],
)


def _combine_body(out0_ref, msg_ref, g_ref, be_ref, o_ref, *, ln):
    h = out0_ref[...] + msg_ref[0] + msg_ref[1]
    if ln:
        mu = jnp.mean(h, axis=-1, keepdims=True)
        var = jnp.mean((h - mu) ** 2, axis=-1, keepdims=True)
        h = (h - mu) / jnp.sqrt(var + 1e-5) * g_ref[...] + be_ref[...]
        h = jnp.maximum(h, 0.0)
    o_ref[...] = h


def _make_combine(ln):
    return pl.pallas_call(
        functools.partial(_combine_body, ln=ln),
        grid=(NP // BD,),
        in_specs=[
            pl.BlockSpec((BD, D), lambda i: (i, 0)),
            pl.BlockSpec((NC, BD, D), lambda i: (0, i, 0)),
            pl.BlockSpec((1, D), lambda i: (0, 0)),
            pl.BlockSpec((1, D), lambda i: (0, 0)),
        ],
        out_specs=pl.BlockSpec((BD, D), lambda i: (i, 0)),
        out_shape=jax.ShapeDtypeStruct((NP, D), f32),
    )


_combine_ln = _make_combine(True)
_combine_plain = _make_combine(False)


def kernel(x, edge_index, edge_type, emb_table, W1, root1, b1, g1, be1,
           W2, root2, b2, g2, be2, W3, root3, b3):
    xi = x.astype(i32)
    src = edge_index[0].astype(i32)
    dst = edge_index[1].astype(i32)
    rel = edge_type.astype(i32)

    xp = jnp.concatenate([xi, jnp.zeros((NP - N_NODES,), i32)])
    padE = EPAD - E
    zpad = jnp.zeros((padE,), i32)
    srcp = jnp.concatenate([src, zpad])
    dstp = jnp.concatenate([dst, jnp.full((padE,), N_NODES, i32)])
    relp = jnp.concatenate([rel, zpad])

    h = _emb_gather(emb_table, xp)
    hist = _edge_hist(dstp, relp)
    inv = _inv_cnt(hist)
    scale, gidx = _edge_scale(srcp, dstp, relp, inv.reshape(G))

    layers = [
        (W1, root1, b1, g1, be1, True),
        (W2, root2, b2, g2, be2, True),
        (W3, root3, b3, None, None, False),
    ]
    for (Wl, rootl, bl, gl, bel, ln) in layers:
        out0, hw = _dense(h, rootl, bl.reshape(1, D), Wl)
        msgs = _msg_pass(hw.reshape(R * NP, DW), gidx, dstp, scale)
        if ln:
            h = _combine_ln(out0, msgs, gl.reshape(1, D), bel.reshape(1, D))
        else:
            h = _combine_plain(out0, msgs, bl.reshape(1, D),
                               bl.reshape(1, D))
    return h[:N_NODES]


# speedup vs baseline: 1.2605x; 1.0242x over previous
import functools

import jax
import jax.numpy as jnp
from jax import lax
from jax.experimental import pallas as pl
from jax.experimental.pallas import tpu as pltpu
from jax.experimental.pallas import tpu_sc as plsc

N_NODES = 10000
NP = 10240
D = 128
R = 8
E = 320000
EPAD = 327680
NC = 2
NS = 16
NW = NC * NS
EPW = EPAD // NW
CB = 128
NCHUNK = EPW // CB
G = R * NP
GR = G // CB
RPS = GR // NS
RPB = NP // NS

f32 = jnp.float32
i32 = jnp.int32

_mesh = functools.partial(
    plsc.VectorSubcoreMesh, core_axis_name="c", subcore_axis_name="s",
    num_cores=NC, num_subcores=NS)

_sc_params = pltpu.CompilerParams(needs_layout_passes=False)
_sc_params_nt = pltpu.CompilerParams(needs_layout_passes=False,
                                     use_tc_tiling_on_sc=False)


def _wid():
    return lax.axis_index("s") * NC + lax.axis_index("c")


@functools.partial(
    pl.kernel,
    out_type=jax.ShapeDtypeStruct((NP, D), f32),
    mesh=_mesh(),
    compiler_params=_sc_params,
    scratch_types=[
        pltpu.VMEM((80,), i32),
        pltpu.VMEM((80, D), f32),
        pltpu.SemaphoreType.DMA,
    ],
)
def _emb_gather(emb_hbm, x_hbm, out_hbm, idx_v, rows_v, sem):
    base = _wid() * (NP // NW)
    for j in range(NP // NW // 80):
        off = base + 80 * j
        pltpu.sync_copy(x_hbm.at[pl.ds(off, 80)], idx_v)
        pltpu.async_copy(emb_hbm.at[idx_v], rows_v, sem).wait()
        pltpu.sync_copy(rows_v, out_hbm.at[pl.ds(off, 80)])


CB0 = 2048


@functools.partial(
    pl.kernel,
    out_type=jax.ShapeDtypeStruct((NW, G), f32),
    mesh=_mesh(),
    compiler_params=_sc_params,
    scratch_types=[
        pltpu.VMEM((G,), f32),
        pltpu.VMEM((CB0,), i32),
        pltpu.VMEM((CB0,), i32),
    ],
)
def _edge_hist(dst_hbm, rel_hbm, hist_hbm, cnt_v, db, rb):
    wid = _wid()
    ones16 = jnp.ones((16,), f32)
    zeros16 = jnp.zeros((16,), f32)

    def zloc(t, _):
        cnt_v[pl.ds(t * 16, 16)] = zeros16
        return 0
    lax.fori_loop(0, G // 16, zloc, 0)

    base_w = wid * EPW

    def cbody(ci, _):
        b = base_w + ci * CB0
        pltpu.sync_copy(dst_hbm.at[pl.ds(b, CB0)], db)
        pltpu.sync_copy(rel_hbm.at[pl.ds(b, CB0)], rb)

        def ib(k, _):
            off = k * 16
            g16 = rb[pl.ds(off, 16)] * NP + db[pl.ds(off, 16)]
            plsc.addupdate_scatter(cnt_v, [g16], ones16)
            return 0
        lax.fori_loop(0, CB0 // 16, ib, 0)
        return 0
    lax.fori_loop(0, EPW // CB0, cbody, 0)
    pltpu.sync_copy(cnt_v, hist_hbm.at[wid])


GB = 8192


def _inv_body(hist_ref, inv_ref):
    s = jnp.sum(hist_ref[...], axis=0, keepdims=True)
    inv_ref[...] = 1.0 / jnp.maximum(s, 1.0)


_inv_cnt = pl.pallas_call(
    _inv_body,
    grid=(G // GB,),
    in_specs=[pl.BlockSpec((NW, GB), lambda i: (0, i))],
    out_specs=pl.BlockSpec((1, GB), lambda i: (0, i)),
    out_shape=jax.ShapeDtypeStruct((1, G), f32),
)


@functools.partial(
    pl.kernel,
    out_type=(jax.ShapeDtypeStruct((EPAD,), f32),
              jax.ShapeDtypeStruct((EPAD,), i32)),
    mesh=_mesh(),
    compiler_params=_sc_params,
    scratch_types=[
        pltpu.VMEM((G,), f32),
        pltpu.VMEM((CB0,), i32),
        pltpu.VMEM((CB0,), i32),
        pltpu.VMEM((CB0,), i32),
        pltpu.VMEM((CB0,), f32),
        pltpu.VMEM((CB0,), i32),
    ],
)
def _edge_scale(src_hbm, dst_hbm, rel_hbm, inv_hbm, scale_hbm, gidx_hbm,
                inv_v, sb, db, rb, scv, gsv):
    wid = _wid()
    iota16 = lax.iota(i32, 16)
    pltpu.sync_copy(inv_hbm, inv_v)
    base_w = wid * EPW

    def cbody(ci, _):
        b = base_w + ci * CB0
        pltpu.sync_copy(src_hbm.at[pl.ds(b, CB0)], sb)
        pltpu.sync_copy(dst_hbm.at[pl.ds(b, CB0)], db)
        pltpu.sync_copy(rel_hbm.at[pl.ds(b, CB0)], rb)

        def ib(k, _):
            off = k * 16
            r16 = rb[pl.ds(off, 16)]
            g16 = r16 * NP + db[pl.ds(off, 16)]
            sc16 = plsc.load_gather(inv_v, [g16])
            eid = b + off + iota16
            scv[pl.ds(off, 16)] = jnp.where(eid < E, sc16, 0.0)
            gsv[pl.ds(off, 16)] = r16 * NP + sb[pl.ds(off, 16)]
            return 0
        lax.fori_loop(0, CB0 // 16, ib, 0)
        pltpu.sync_copy(scv, scale_hbm.at[pl.ds(b, CB0)])
        pltpu.sync_copy(gsv, gidx_hbm.at[pl.ds(b, CB0)])
        return 0
    lax.fori_loop(0, EPW // CB0, cbody, 0)


DW = D // 2
WCH = 16
C0 = 80
C1 = 160 - C0


@functools.partial(
    pl.kernel,
    out_type=jax.ShapeDtypeStruct((NC, NP, D), f32),
    mesh=_mesh(),
    compiler_params=_sc_params_nt,
    scratch_types=[
        pltpu.VMEM((WCH * CB,), i32),
        pltpu.VMEM((WCH * CB,), i32),
        pltpu.VMEM((WCH * CB,), f32),
        pltpu.VMEM((CB,), i32),
        pltpu.VMEM((CB,), i32),
        pltpu.VMEM((CB,), i32),
        pltpu.VMEM((CB,), i32),
        pltpu.VMEM((CB, DW), i32),
        pltpu.VMEM((CB, DW), i32),
        pltpu.VMEM((CB, D), f32),
        pltpu.VMEM_SHARED((NP, D), f32),
        pltpu.SemaphoreType.DMA,
        pltpu.SemaphoreType.DMA,
    ],
)
def _msg_pass(hw_hbm, gidx_hbm, dst_hbm, scale_hbm, out_hbm,
              ga, da, scv, gc0, gc1, dc0, dc1, brows0, brows1, srows, acc,
              sem0, sem1):
    cid = lax.axis_index("c")
    sid = lax.axis_index("s")
    zeros16 = jnp.zeros((16,), f32)
    himask = jnp.full((16,), -65536, i32)
    brows = (brows0, brows1)
    gcs = (gc0, gc1)
    dcs = (dc0, dc1)
    sems = (sem0, sem1)

    def zrow(t, _):
        for k in range(D // 16):
            srows[t, pl.ds(k * 16, 16)] = zeros16
        return 0
    lax.fori_loop(0, CB, zrow, 0)
    for j in range(RPB // CB):
        pltpu.sync_copy(srows, acc.at[pl.ds(sid * RPB + j * CB, CB)])
    plsc.subcore_barrier()

    def fill(dst_ref, src_ref, lc):
        for k in range(CB // 16):
            dst_ref[pl.ds(k * 16, 16)] = src_ref[pl.ds(lc * CB + k * 16, 16)]

    def do_chunk(lc, b):
        nxt = lc + 1

        @pl.when(nxt < WCH)
        def _():
            fill(gcs[1 - b], ga, nxt)
            pltpu.async_copy(hw_hbm.at[gcs[1 - b]], brows[1 - b],
                             sems[1 - b])
        pltpu.make_async_copy(hw_hbm.at[gcs[b]], brows[b], sems[b]).wait()
        sbase = lc * CB

        def srow(jj, _):
            for u in range(4):
                j = jj * 4 + u
                sj = plsc.load_gather(scv, [jnp.full((16,), sbase + j, i32)])
                for k in range(DW // 16):
                    v = brows[b][j, pl.ds(16 * k, 16)]
                    lo = plsc.bitcast(v << 16, f32)
                    hi = plsc.bitcast(v & himask, f32)
                    srows[j, pl.ds(16 * k, 16)] = lo * sj
                    srows[j, pl.ds(64 + 16 * k, 16)] = hi * sj
            return 0
        lax.fori_loop(0, CB // 4, srow, 0)
        fill(dcs[b], da, lc)
        pltpu.sync_copy(srows, acc.at[dcs[b]], add=True)

    base_e = (sid * (C0 + C1) + cid * C0) * CB
    nwin = (C0 - (C0 - C1) * cid) // WCH

    def window(w, _):
        eb = base_e + w * (WCH * CB)
        pltpu.sync_copy(gidx_hbm.at[pl.ds(eb, WCH * CB)], ga)
        pltpu.sync_copy(dst_hbm.at[pl.ds(eb, WCH * CB)], da)
        pltpu.sync_copy(scale_hbm.at[pl.ds(eb, WCH * CB)], scv)
        fill(gc0, ga, 0)
        pltpu.async_copy(hw_hbm.at[gc0], brows0, sem0)

        def pair(i, _):
            do_chunk(2 * i, 0)
            do_chunk(2 * i + 1, 1)
            return 0
        lax.fori_loop(0, WCH // 2, pair, 0)
        return 0
    lax.fori_loop(0, nwin, window, 0)
    plsc.subcore_barrier()

    for j in range(RPB // CB):
        rs = sid * RPB + j * CB
        pltpu.sync_copy(acc.at[pl.ds(rs, CB)], out_hbm.at[cid, pl.ds(rs, CB)])

BD = 1024


def _dense_body(h_ref, root_ref, b_ref, w_ref, out0_ref, hw_ref):
    x = h_ref[...]
    out0_ref[...] = (
        jnp.dot(x, root_ref[...], preferred_element_type=f32) + b_ref[...])
    for r in range(R):
        y = jnp.dot(x, w_ref[r], preferred_element_type=f32)
        yb = y.astype(jnp.bfloat16).astype(f32)
        bits = lax.bitcast_convert_type(yb, jnp.uint32)
        word = (bits[:, :DW] >> 16) | (bits[:, DW:] & jnp.uint32(0xFFFF0000))
        hw_ref[r] = lax.bitcast_convert_type(word, i32)


_dense = pl.pallas_call(
    _dense_body,
    grid=(NP // BD,),
    in_specs=[
        pl.BlockSpec((BD, D), lambda i: (i, 0)),
        pl.BlockSpec((D, D), lambda i: (0, 0)),
        pl.BlockSpec((1, D), lambda i: (0, 0)),
        pl.BlockSpec((R, D, D), lambda i: (0, 0, 0)),
    ],
    out_specs=[
        pl.BlockSpec((BD, D), lambda i: (i, 0)),
        pl.BlockSpec((R, BD, DW), lambda i: (0, i, 0)),
    ],
    out_shape=[
        jax.ShapeDtypeStruct((NP, D), f32),
        jax.ShapeDtypeStruct((R, NP, DW), i32),
    ],
)


def _combine_body(out0_ref, msg_ref, g_ref, be_ref, o_ref, *, ln):
    h = out0_ref[...] + msg_ref[0] + msg_ref[1]
    if ln:
        mu = jnp.mean(h, axis=-1, keepdims=True)
        var = jnp.mean((h - mu) ** 2, axis=-1, keepdims=True)
        h = (h - mu) / jnp.sqrt(var + 1e-5) * g_ref[...] + be_ref[...]
        h = jnp.maximum(h, 0.0)
    o_ref[...] = h


def _make_combine(ln):
    return pl.pallas_call(
        functools.partial(_combine_body, ln=ln),
        grid=(NP // BD,),
        in_specs=[
            pl.BlockSpec((BD, D), lambda i: (i, 0)),
            pl.BlockSpec((NC, BD, D), lambda i: (0, i, 0)),
            pl.BlockSpec((1, D), lambda i: (0, 0)),
            pl.BlockSpec((1, D), lambda i: (0, 0)),
        ],
        out_specs=pl.BlockSpec((BD, D), lambda i: (i, 0)),
        out_shape=jax.ShapeDtypeStruct((NP, D), f32),
    )


_combine_ln = _make_combine(True)
_combine_plain = _make_combine(False)


def kernel(x, edge_index, edge_type, emb_table, W1, root1, b1, g1, be1,
           W2, root2, b2, g2, be2, W3, root3, b3):
    xi = x.astype(i32)
    src = edge_index[0].astype(i32)
    dst = edge_index[1].astype(i32)
    rel = edge_type.astype(i32)

    xp = jnp.concatenate([xi, jnp.zeros((NP - N_NODES,), i32)])
    padE = EPAD - E
    zpad = jnp.zeros((padE,), i32)
    srcp = jnp.concatenate([src, zpad])
    dstp = jnp.concatenate([dst, jnp.full((padE,), N_NODES, i32)])
    relp = jnp.concatenate([rel, zpad])

    h = _emb_gather(emb_table, xp)
    hist = _edge_hist(dstp, relp)
    inv = _inv_cnt(hist)
    scale, gidx = _edge_scale(srcp, dstp, relp, inv.reshape(G))

    layers = [
        (W1, root1, b1, g1, be1, True),
        (W2, root2, b2, g2, be2, True),
        (W3, root3, b3, None, None, False),
    ]
    for (Wl, rootl, bl, gl, bel, ln) in layers:
        out0, hw = _dense(h, rootl, bl.reshape(1, D), Wl)
        msgs = _msg_pass(hw.reshape(R * NP, DW), gidx, dstp, scale)
        if ln:
            h = _combine_ln(out0, msgs, gl.reshape(1, D), bel.reshape(1, D))
        else:
            h = _combine_plain(out0, msgs, bl.reshape(1, D),
                               bl.reshape(1, D))
    return h[:N_NODES]


# speedup vs baseline: 1.2620x; 1.0012x over previous
import functools

import jax
import jax.numpy as jnp
from jax import lax
from jax.experimental import pallas as pl
from jax.experimental.pallas import tpu as pltpu
from jax.experimental.pallas import tpu_sc as plsc

N_NODES = 10000
NP = 10240
D = 128
R = 8
E = 320000
EPAD = 327680
NC = 2
NS = 16
NW = NC * NS
EPW = EPAD // NW
CB = 128
NCHUNK = EPW // CB
G = R * NP
GR = G // CB
RPS = GR // NS
RPB = NP // NS

f32 = jnp.float32
i32 = jnp.int32

_mesh = functools.partial(
    plsc.VectorSubcoreMesh, core_axis_name="c", subcore_axis_name="s",
    num_cores=NC, num_subcores=NS)

_sc_params = pltpu.CompilerParams(needs_layout_passes=False)
_sc_params_nt = pltpu.CompilerParams(needs_layout_passes=False,
                                     use_tc_tiling_on_sc=False)


def _wid():
    return lax.axis_index("s") * NC + lax.axis_index("c")


@functools.partial(
    pl.kernel,
    out_type=jax.ShapeDtypeStruct((NP, D), f32),
    mesh=_mesh(),
    compiler_params=_sc_params,
    scratch_types=[
        pltpu.VMEM((80,), i32),
        pltpu.VMEM((80, D), f32),
        pltpu.SemaphoreType.DMA,
    ],
)
def _emb_gather(emb_hbm, x_hbm, out_hbm, idx_v, rows_v, sem):
    base = _wid() * (NP // NW)
    for j in range(NP // NW // 80):
        off = base + 80 * j
        pltpu.sync_copy(x_hbm.at[pl.ds(off, 80)], idx_v)
        pltpu.async_copy(emb_hbm.at[idx_v], rows_v, sem).wait()
        pltpu.sync_copy(rows_v, out_hbm.at[pl.ds(off, 80)])


CB0 = 2048


@functools.partial(
    pl.kernel,
    out_type=jax.ShapeDtypeStruct((NW, G), f32),
    mesh=_mesh(),
    compiler_params=_sc_params,
    scratch_types=[
        pltpu.VMEM((G,), f32),
        pltpu.VMEM((CB0,), i32),
        pltpu.VMEM((CB0,), i32),
    ],
)
def _edge_hist(dst_hbm, rel_hbm, hist_hbm, cnt_v, db, rb):
    wid = _wid()
    ones16 = jnp.ones((16,), f32)
    zeros16 = jnp.zeros((16,), f32)

    def zloc(t, _):
        cnt_v[pl.ds(t * 16, 16)] = zeros16
        return 0
    lax.fori_loop(0, G // 16, zloc, 0)

    base_w = wid * EPW

    def cbody(ci, _):
        b = base_w + ci * CB0
        pltpu.sync_copy(dst_hbm.at[pl.ds(b, CB0)], db)
        pltpu.sync_copy(rel_hbm.at[pl.ds(b, CB0)], rb)

        def ib(k, _):
            off = k * 16
            g16 = rb[pl.ds(off, 16)] * NP + db[pl.ds(off, 16)]
            plsc.addupdate_scatter(cnt_v, [g16], ones16)
            return 0
        lax.fori_loop(0, CB0 // 16, ib, 0)
        return 0
    lax.fori_loop(0, EPW // CB0, cbody, 0)
    pltpu.sync_copy(cnt_v, hist_hbm.at[wid])


GB = 8192


def _inv_body(hist_ref, inv_ref):
    s = jnp.sum(hist_ref[...], axis=0, keepdims=True)
    inv_ref[...] = 1.0 / jnp.maximum(s, 1.0)


_inv_cnt = pl.pallas_call(
    _inv_body,
    grid=(G // GB,),
    in_specs=[pl.BlockSpec((NW, GB), lambda i: (0, i))],
    out_specs=pl.BlockSpec((1, GB), lambda i: (0, i)),
    out_shape=jax.ShapeDtypeStruct((1, G), f32),
)


@functools.partial(
    pl.kernel,
    out_type=(jax.ShapeDtypeStruct((EPAD,), f32),
              jax.ShapeDtypeStruct((EPAD,), i32)),
    mesh=_mesh(),
    compiler_params=_sc_params,
    scratch_types=[
        pltpu.VMEM((G,), f32),
        pltpu.VMEM((CB0,), i32),
        pltpu.VMEM((CB0,), i32),
        pltpu.VMEM((CB0,), i32),
        pltpu.VMEM((CB0,), f32),
        pltpu.VMEM((CB0,), i32),
    ],
)
def _edge_scale(src_hbm, dst_hbm, rel_hbm, inv_hbm, scale_hbm, gidx_hbm,
                inv_v, sb, db, rb, scv, gsv):
    wid = _wid()
    iota16 = lax.iota(i32, 16)
    pltpu.sync_copy(inv_hbm, inv_v)
    base_w = wid * EPW

    def cbody(ci, _):
        b = base_w + ci * CB0
        pltpu.sync_copy(src_hbm.at[pl.ds(b, CB0)], sb)
        pltpu.sync_copy(dst_hbm.at[pl.ds(b, CB0)], db)
        pltpu.sync_copy(rel_hbm.at[pl.ds(b, CB0)], rb)

        def ib(k, _):
            off = k * 16
            r16 = rb[pl.ds(off, 16)]
            g16 = r16 * NP + db[pl.ds(off, 16)]
            sc16 = plsc.load_gather(inv_v, [g16])
            eid = b + off + iota16
            scv[pl.ds(off, 16)] = jnp.where(eid < E, sc16, 0.0)
            gsv[pl.ds(off, 16)] = r16 * NP + sb[pl.ds(off, 16)]
            return 0
        lax.fori_loop(0, CB0 // 16, ib, 0)
        pltpu.sync_copy(scv, scale_hbm.at[pl.ds(b, CB0)])
        pltpu.sync_copy(gsv, gidx_hbm.at[pl.ds(b, CB0)])
        return 0
    lax.fori_loop(0, EPW // CB0, cbody, 0)


DW = D // 2
WCH = 16
C0 = 80
C1 = 160 - C0


@functools.partial(
    pl.kernel,
    out_type=jax.ShapeDtypeStruct((NC, NP, D), f32),
    mesh=_mesh(),
    compiler_params=_sc_params_nt,
    scratch_types=[
        pltpu.VMEM((WCH * CB,), i32),
        pltpu.VMEM((WCH * CB,), i32),
        pltpu.VMEM((WCH * CB,), f32),
        pltpu.VMEM((CB,), i32),
        pltpu.VMEM((CB,), i32),
        pltpu.VMEM((CB,), i32),
        pltpu.VMEM((CB,), i32),
        pltpu.VMEM((CB, DW), i32),
        pltpu.VMEM((CB, DW), i32),
        pltpu.VMEM((CB, D), f32),
        pltpu.VMEM_SHARED((NP, D), f32),
        pltpu.SemaphoreType.DMA,
        pltpu.SemaphoreType.DMA,
    ],
)
def _msg_pass(hw_hbm, gidx_hbm, dst_hbm, scale_hbm, out_hbm,
              ga, da, scv, gc0, gc1, dc0, dc1, brows0, brows1, srows, acc,
              sem0, sem1):
    cid = lax.axis_index("c")
    sid = lax.axis_index("s")
    zeros16 = jnp.zeros((16,), f32)
    himask = jnp.full((16,), -65536, i32)
    brows = (brows0, brows1)
    gcs = (gc0, gc1)
    dcs = (dc0, dc1)
    sems = (sem0, sem1)

    def zrow(t, _):
        for k in range(D // 16):
            srows[t, pl.ds(k * 16, 16)] = zeros16
        return 0
    lax.fori_loop(0, CB, zrow, 0)
    for j in range(RPB // CB):
        pltpu.sync_copy(srows, acc.at[pl.ds(sid * RPB + j * CB, CB)])
    plsc.subcore_barrier()

    def fill(dst_ref, src_ref, lc):
        for k in range(CB // 16):
            dst_ref[pl.ds(k * 16, 16)] = src_ref[pl.ds(lc * CB + k * 16, 16)]

    def do_chunk(lc, b):
        nxt = lc + 1

        @pl.when(nxt < WCH)
        def _():
            fill(gcs[1 - b], ga, nxt)
            pltpu.async_copy(hw_hbm.at[gcs[1 - b]], brows[1 - b],
                             sems[1 - b])
        pltpu.make_async_copy(hw_hbm.at[gcs[b]], brows[b], sems[b]).wait()
        sbase = lc * CB

        def srow(jj, _):
            for u in range(8):
                j = jj * 8 + u
                sj = plsc.load_gather(scv, [jnp.full((16,), sbase + j, i32)])
                for k in range(DW // 16):
                    v = brows[b][j, pl.ds(16 * k, 16)]
                    lo = plsc.bitcast(v << 16, f32)
                    hi = plsc.bitcast(v & himask, f32)
                    srows[j, pl.ds(16 * k, 16)] = lo * sj
                    srows[j, pl.ds(64 + 16 * k, 16)] = hi * sj
            return 0
        lax.fori_loop(0, CB // 8, srow, 0)
        fill(dcs[b], da, lc)
        pltpu.sync_copy(srows, acc.at[dcs[b]], add=True)

    base_e = (sid * (C0 + C1) + cid * C0) * CB
    nwin = (C0 - (C0 - C1) * cid) // WCH

    def window(w, _):
        eb = base_e + w * (WCH * CB)
        pltpu.sync_copy(gidx_hbm.at[pl.ds(eb, WCH * CB)], ga)
        pltpu.sync_copy(dst_hbm.at[pl.ds(eb, WCH * CB)], da)
        pltpu.sync_copy(scale_hbm.at[pl.ds(eb, WCH * CB)], scv)
        fill(gc0, ga, 0)
        pltpu.async_copy(hw_hbm.at[gc0], brows0, sem0)

        def pair(i, _):
            do_chunk(2 * i, 0)
            do_chunk(2 * i + 1, 1)
            return 0
        lax.fori_loop(0, WCH // 2, pair, 0)
        return 0
    lax.fori_loop(0, nwin, window, 0)
    plsc.subcore_barrier()

    for j in range(RPB // CB):
        rs = sid * RPB + j * CB
        pltpu.sync_copy(acc.at[pl.ds(rs, CB)], out_hbm.at[cid, pl.ds(rs, CB)])

BD = 1024


def _dense_body(h_ref, root_ref, b_ref, w_ref, out0_ref, hw_ref):
    x = h_ref[...]
    out0_ref[...] = (
        jnp.dot(x, root_ref[...], preferred_element_type=f32) + b_ref[...])
    for r in range(R):
        y = jnp.dot(x, w_ref[r], preferred_element_type=f32)
        yb = y.astype(jnp.bfloat16).astype(f32)
        bits = lax.bitcast_convert_type(yb, jnp.uint32)
        word = (bits[:, :DW] >> 16) | (bits[:, DW:] & jnp.uint32(0xFFFF0000))
        hw_ref[r] = lax.bitcast_convert_type(word, i32)


_dense = pl.pallas_call(
    _dense_body,
    grid=(NP // BD,),
    in_specs=[
        pl.BlockSpec((BD, D), lambda i: (i, 0)),
        pl.BlockSpec((D, D), lambda i: (0, 0)),
        pl.BlockSpec((1, D), lambda i: (0, 0)),
        pl.BlockSpec((R, D, D), lambda i: (0, 0, 0)),
    ],
    out_specs=[
        pl.BlockSpec((BD, D), lambda i: (i, 0)),
        pl.BlockSpec((R, BD, DW), lambda i: (0, i, 0)),
    ],
    out_shape=[
        jax.ShapeDtypeStruct((NP, D), f32),
        jax.ShapeDtypeStruct((R, NP, DW), i32),
    ],
)


def _combine_body(out0_ref, msg_ref, g_ref, be_ref, o_ref, *, ln):
    h = out0_ref[...] + msg_ref[0] + msg_ref[1]
    if ln:
        mu = jnp.mean(h, axis=-1, keepdims=True)
        var = jnp.mean((h - mu) ** 2, axis=-1, keepdims=True)
        h = (h - mu) / jnp.sqrt(var + 1e-5) * g_ref[...] + be_ref[...]
        h = jnp.maximum(h, 0.0)
    o_ref[...] = h


def _make_combine(ln):
    return pl.pallas_call(
        functools.partial(_combine_body, ln=ln),
        grid=(NP // BD,),
        in_specs=[
            pl.BlockSpec((BD, D), lambda i: (i, 0)),
            pl.BlockSpec((NC, BD, D), lambda i: (0, i, 0)),
            pl.BlockSpec((1, D), lambda i: (0, 0)),
            pl.BlockSpec((1, D), lambda i: (0, 0)),
        ],
        out_specs=pl.BlockSpec((BD, D), lambda i: (i, 0)),
        out_shape=jax.ShapeDtypeStruct((NP, D), f32),
    )


_combine_ln = _make_combine(True)
_combine_plain = _make_combine(False)


def kernel(x, edge_index, edge_type, emb_table, W1, root1, b1, g1, be1,
           W2, root2, b2, g2, be2, W3, root3, b3):
    xi = x.astype(i32)
    src = edge_index[0].astype(i32)
    dst = edge_index[1].astype(i32)
    rel = edge_type.astype(i32)

    xp = jnp.concatenate([xi, jnp.zeros((NP - N_NODES,), i32)])
    padE = EPAD - E
    zpad = jnp.zeros((padE,), i32)
    srcp = jnp.concatenate([src, zpad])
    dstp = jnp.concatenate([dst, jnp.full((padE,), N_NODES, i32)])
    relp = jnp.concatenate([rel, zpad])

    h = _emb_gather(emb_table, xp)
    hist = _edge_hist(dstp, relp)
    inv = _inv_cnt(hist)
    scale, gidx = _edge_scale(srcp, dstp, relp, inv.reshape(G))

    layers = [
        (W1, root1, b1, g1, be1, True),
        (W2, root2, b2, g2, be2, True),
        (W3, root3, b3, None, None, False),
    ]
    for (Wl, rootl, bl, gl, bel, ln) in layers:
        out0, hw = _dense(h, rootl, bl.reshape(1, D), Wl)
        msgs = _msg_pass(hw.reshape(R * NP, DW), gidx, dstp, scale)
        if ln:
            h = _combine_ln(out0, msgs, gl.reshape(1, D), bel.reshape(1, D))
        else:
            h = _combine_plain(out0, msgs, bl.reshape(1, D),
                               bl.reshape(1, D))
    return h[:N_NODES]


# speedup vs baseline: 1.2685x; 1.0051x over previous
import functools

import jax
import jax.numpy as jnp
from jax import lax
from jax.experimental import pallas as pl
from jax.experimental.pallas import tpu as pltpu
from jax.experimental.pallas import tpu_sc as plsc

N_NODES = 10000
NP = 10240
D = 128
R = 8
E = 320000
EPAD = 327680
NC = 2
NS = 16
NW = NC * NS
EPW = EPAD // NW
CB = 128
NCHUNK = EPW // CB
G = R * NP
GR = G // CB
RPS = GR // NS
RPB = NP // NS

f32 = jnp.float32
i32 = jnp.int32

_mesh = functools.partial(
    plsc.VectorSubcoreMesh, core_axis_name="c", subcore_axis_name="s",
    num_cores=NC, num_subcores=NS)

_sc_params = pltpu.CompilerParams(needs_layout_passes=False)
_sc_params_nt = pltpu.CompilerParams(needs_layout_passes=False,
                                     use_tc_tiling_on_sc=False)


def _wid():
    return lax.axis_index("s") * NC + lax.axis_index("c")


@functools.partial(
    pl.kernel,
    out_type=jax.ShapeDtypeStruct((NP, D), f32),
    mesh=_mesh(),
    compiler_params=_sc_params,
    scratch_types=[
        pltpu.VMEM((80,), i32),
        pltpu.VMEM((80, D), f32),
        pltpu.SemaphoreType.DMA,
    ],
)
def _emb_gather(emb_hbm, x_hbm, out_hbm, idx_v, rows_v, sem):
    base = _wid() * (NP // NW)
    for j in range(NP // NW // 80):
        off = base + 80 * j
        pltpu.sync_copy(x_hbm.at[pl.ds(off, 80)], idx_v)
        pltpu.async_copy(emb_hbm.at[idx_v], rows_v, sem).wait()
        pltpu.sync_copy(rows_v, out_hbm.at[pl.ds(off, 80)])


CB0 = 2048


@functools.partial(
    pl.kernel,
    out_type=jax.ShapeDtypeStruct((NW, G), f32),
    mesh=_mesh(),
    compiler_params=_sc_params,
    scratch_types=[
        pltpu.VMEM((G,), f32),
        pltpu.VMEM((CB0,), i32),
        pltpu.VMEM((CB0,), i32),
    ],
)
def _edge_hist(dst_hbm, rel_hbm, hist_hbm, cnt_v, db, rb):
    wid = _wid()
    ones16 = jnp.ones((16,), f32)
    zeros16 = jnp.zeros((16,), f32)

    def zloc(t, _):
        cnt_v[pl.ds(t * 16, 16)] = zeros16
        return 0
    lax.fori_loop(0, G // 16, zloc, 0)

    base_w = wid * EPW

    def cbody(ci, _):
        b = base_w + ci * CB0
        pltpu.sync_copy(dst_hbm.at[pl.ds(b, CB0)], db)
        pltpu.sync_copy(rel_hbm.at[pl.ds(b, CB0)], rb)

        def ib(k, _):
            off = k * 16
            g16 = rb[pl.ds(off, 16)] * NP + db[pl.ds(off, 16)]
            plsc.addupdate_scatter(cnt_v, [g16], ones16)
            return 0
        lax.fori_loop(0, CB0 // 16, ib, 0)
        return 0
    lax.fori_loop(0, EPW // CB0, cbody, 0)
    pltpu.sync_copy(cnt_v, hist_hbm.at[wid])


GB = 8192


def _inv_body(hist_ref, inv_ref):
    s = jnp.sum(hist_ref[...], axis=0, keepdims=True)
    inv_ref[...] = 1.0 / jnp.maximum(s, 1.0)


_inv_cnt = pl.pallas_call(
    _inv_body,
    grid=(G // GB,),
    in_specs=[pl.BlockSpec((NW, GB), lambda i: (0, i))],
    out_specs=pl.BlockSpec((1, GB), lambda i: (0, i)),
    out_shape=jax.ShapeDtypeStruct((1, G), f32),
)


@functools.partial(
    pl.kernel,
    out_type=(jax.ShapeDtypeStruct((EPAD,), f32),
              jax.ShapeDtypeStruct((EPAD,), i32)),
    mesh=_mesh(),
    compiler_params=_sc_params,
    scratch_types=[
        pltpu.VMEM((G,), f32),
        pltpu.VMEM((CB0,), i32),
        pltpu.VMEM((CB0,), i32),
        pltpu.VMEM((CB0,), i32),
        pltpu.VMEM((CB0,), f32),
        pltpu.VMEM((CB0,), i32),
    ],
)
def _edge_scale(src_hbm, dst_hbm, rel_hbm, inv_hbm, scale_hbm, gidx_hbm,
                inv_v, sb, db, rb, scv, gsv):
    wid = _wid()
    iota16 = lax.iota(i32, 16)
    pltpu.sync_copy(inv_hbm, inv_v)
    base_w = wid * EPW

    def cbody(ci, _):
        b = base_w + ci * CB0
        pltpu.sync_copy(src_hbm.at[pl.ds(b, CB0)], sb)
        pltpu.sync_copy(dst_hbm.at[pl.ds(b, CB0)], db)
        pltpu.sync_copy(rel_hbm.at[pl.ds(b, CB0)], rb)

        def ib(k, _):
            off = k * 16
            r16 = rb[pl.ds(off, 16)]
            g16 = r16 * NP + db[pl.ds(off, 16)]
            sc16 = plsc.load_gather(inv_v, [g16])
            eid = b + off + iota16
            scv[pl.ds(off, 16)] = jnp.where(eid < E, sc16, 0.0)
            gsv[pl.ds(off, 16)] = r16 * NP + sb[pl.ds(off, 16)]
            return 0
        lax.fori_loop(0, CB0 // 16, ib, 0)
        pltpu.sync_copy(scv, scale_hbm.at[pl.ds(b, CB0)])
        pltpu.sync_copy(gsv, gidx_hbm.at[pl.ds(b, CB0)])
        return 0
    lax.fori_loop(0, EPW // CB0, cbody, 0)


DW = D // 2
WCH = 20
C0 = 80
C1 = 160 - C0


@functools.partial(
    pl.kernel,
    out_type=jax.ShapeDtypeStruct((NC, NP, D), f32),
    mesh=_mesh(),
    compiler_params=_sc_params_nt,
    scratch_types=[
        pltpu.VMEM((WCH * CB,), i32),
        pltpu.VMEM((WCH * CB,), i32),
        pltpu.VMEM((WCH * CB,), f32),
        pltpu.VMEM((CB,), i32),
        pltpu.VMEM((CB,), i32),
        pltpu.VMEM((CB,), i32),
        pltpu.VMEM((CB,), i32),
        pltpu.VMEM((CB, DW), i32),
        pltpu.VMEM((CB, DW), i32),
        pltpu.VMEM((CB, D), f32),
        pltpu.VMEM_SHARED((NP, D), f32),
        pltpu.SemaphoreType.DMA,
        pltpu.SemaphoreType.DMA,
    ],
)
def _msg_pass(hw_hbm, gidx_hbm, dst_hbm, scale_hbm, out_hbm,
              ga, da, scv, gc0, gc1, dc0, dc1, brows0, brows1, srows, acc,
              sem0, sem1):
    cid = lax.axis_index("c")
    sid = lax.axis_index("s")
    zeros16 = jnp.zeros((16,), f32)
    himask = jnp.full((16,), -65536, i32)
    brows = (brows0, brows1)
    gcs = (gc0, gc1)
    dcs = (dc0, dc1)
    sems = (sem0, sem1)

    def zrow(t, _):
        for k in range(D // 16):
            srows[t, pl.ds(k * 16, 16)] = zeros16
        return 0
    lax.fori_loop(0, CB, zrow, 0)
    for j in range(RPB // CB):
        pltpu.sync_copy(srows, acc.at[pl.ds(sid * RPB + j * CB, CB)])
    plsc.subcore_barrier()

    def fill(dst_ref, src_ref, lc):
        for k in range(CB // 16):
            dst_ref[pl.ds(k * 16, 16)] = src_ref[pl.ds(lc * CB + k * 16, 16)]

    def do_chunk(lc, b):
        nxt = lc + 1

        @pl.when(nxt < WCH)
        def _():
            fill(gcs[1 - b], ga, nxt)
            pltpu.async_copy(hw_hbm.at[gcs[1 - b]], brows[1 - b],
                             sems[1 - b])
        pltpu.make_async_copy(hw_hbm.at[gcs[b]], brows[b], sems[b]).wait()
        sbase = lc * CB

        def srow(jj, _):
            for u in range(8):
                j = jj * 8 + u
                sj = plsc.load_gather(scv, [jnp.full((16,), sbase + j, i32)])
                for k in range(DW // 16):
                    v = brows[b][j, pl.ds(16 * k, 16)]
                    lo = plsc.bitcast(v << 16, f32)
                    hi = plsc.bitcast(v & himask, f32)
                    srows[j, pl.ds(16 * k, 16)] = lo * sj
                    srows[j, pl.ds(64 + 16 * k, 16)] = hi * sj
            return 0
        lax.fori_loop(0, CB // 8, srow, 0)
        fill(dcs[b], da, lc)
        pltpu.sync_copy(srows, acc.at[dcs[b]], add=True)

    base_e = (sid * (C0 + C1) + cid * C0) * CB
    nwin = (C0 - (C0 - C1) * cid) // WCH

    def window(w, _):
        eb = base_e + w * (WCH * CB)
        pltpu.sync_copy(gidx_hbm.at[pl.ds(eb, WCH * CB)], ga)
        pltpu.sync_copy(dst_hbm.at[pl.ds(eb, WCH * CB)], da)
        pltpu.sync_copy(scale_hbm.at[pl.ds(eb, WCH * CB)], scv)
        fill(gc0, ga, 0)
        pltpu.async_copy(hw_hbm.at[gc0], brows0, sem0)

        def pair(i, _):
            do_chunk(2 * i, 0)
            do_chunk(2 * i + 1, 1)
            return 0
        lax.fori_loop(0, WCH // 2, pair, 0)
        return 0
    lax.fori_loop(0, nwin, window, 0)
    plsc.subcore_barrier()

    for j in range(RPB // CB):
        rs = sid * RPB + j * CB
        pltpu.sync_copy(acc.at[pl.ds(rs, CB)], out_hbm.at[cid, pl.ds(rs, CB)])

BD = 1024


def _dense_body(h_ref, root_ref, b_ref, w_ref, out0_ref, hw_ref):
    x = h_ref[...]
    out0_ref[...] = (
        jnp.dot(x, root_ref[...], preferred_element_type=f32) + b_ref[...])
    for r in range(R):
        y = jnp.dot(x, w_ref[r], preferred_element_type=f32)
        yb = y.astype(jnp.bfloat16).astype(f32)
        bits = lax.bitcast_convert_type(yb, jnp.uint32)
        word = (bits[:, :DW] >> 16) | (bits[:, DW:] & jnp.uint32(0xFFFF0000))
        hw_ref[r] = lax.bitcast_convert_type(word, i32)


_dense = pl.pallas_call(
    _dense_body,
    grid=(NP // BD,),
    in_specs=[
        pl.BlockSpec((BD, D), lambda i: (i, 0)),
        pl.BlockSpec((D, D), lambda i: (0, 0)),
        pl.BlockSpec((1, D), lambda i: (0, 0)),
        pl.BlockSpec((R, D, D), lambda i: (0, 0, 0)),
    ],
    out_specs=[
        pl.BlockSpec((BD, D), lambda i: (i, 0)),
        pl.BlockSpec((R, BD, DW), lambda i: (0, i, 0)),
    ],
    out_shape=[
        jax.ShapeDtypeStruct((NP, D), f32),
        jax.ShapeDtypeStruct((R, NP, DW), i32),
    ],
)


def _combine_body(out0_ref, msg_ref, g_ref, be_ref, o_ref, *, ln):
    h = out0_ref[...] + msg_ref[0] + msg_ref[1]
    if ln:
        mu = jnp.mean(h, axis=-1, keepdims=True)
        var = jnp.mean((h - mu) ** 2, axis=-1, keepdims=True)
        h = (h - mu) / jnp.sqrt(var + 1e-5) * g_ref[...] + be_ref[...]
        h = jnp.maximum(h, 0.0)
    o_ref[...] = h


def _make_combine(ln):
    return pl.pallas_call(
        functools.partial(_combine_body, ln=ln),
        grid=(NP // BD,),
        in_specs=[
            pl.BlockSpec((BD, D), lambda i: (i, 0)),
            pl.BlockSpec((NC, BD, D), lambda i: (0, i, 0)),
            pl.BlockSpec((1, D), lambda i: (0, 0)),
            pl.BlockSpec((1, D), lambda i: (0, 0)),
        ],
        out_specs=pl.BlockSpec((BD, D), lambda i: (i, 0)),
        out_shape=jax.ShapeDtypeStruct((NP, D), f32),
    )


_combine_ln = _make_combine(True)
_combine_plain = _make_combine(False)


def kernel(x, edge_index, edge_type, emb_table, W1, root1, b1, g1, be1,
           W2, root2, b2, g2, be2, W3, root3, b3):
    xi = x.astype(i32)
    src = edge_index[0].astype(i32)
    dst = edge_index[1].astype(i32)
    rel = edge_type.astype(i32)

    xp = jnp.concatenate([xi, jnp.zeros((NP - N_NODES,), i32)])
    padE = EPAD - E
    zpad = jnp.zeros((padE,), i32)
    srcp = jnp.concatenate([src, zpad])
    dstp = jnp.concatenate([dst, jnp.full((padE,), N_NODES, i32)])
    relp = jnp.concatenate([rel, zpad])

    h = _emb_gather(emb_table, xp)
    hist = _edge_hist(dstp, relp)
    inv = _inv_cnt(hist)
    scale, gidx = _edge_scale(srcp, dstp, relp, inv.reshape(G))

    layers = [
        (W1, root1, b1, g1, be1, True),
        (W2, root2, b2, g2, be2, True),
        (W3, root3, b3, None, None, False),
    ]
    for (Wl, rootl, bl, gl, bel, ln) in layers:
        out0, hw = _dense(h, rootl, bl.reshape(1, D), Wl)
        msgs = _msg_pass(hw.reshape(R * NP, DW), gidx, dstp, scale)
        if ln:
            h = _combine_ln(out0, msgs, gl.reshape(1, D), bel.reshape(1, D))
        else:
            h = _combine_plain(out0, msgs, bl.reshape(1, D),
                               bl.reshape(1, D))
    return h[:N_NODES]
